# baseline (speedup 1.0000x reference)
import functools

import jax
import jax.numpy as jnp
from jax import lax
from jax.experimental import pallas as pl
from jax.experimental.pallas import tpu as pltpu
from jax.experimental.pallas import tpu_sc as plsc

NUM_ENTITIES = 1000000
NUM_RELATIONS = 1000
EMB_DIM = 32
BATCH = 16384

NC = 2
NS = 16
NW = NC * NS
BPW = BATCH // NW
CH = 128
NCH = BPW // CH
WIDE = 128
RPW = EMB_DIM // 16


def _prep_indices(idx_ref, j_ref, off_ref):
  for k in range(NCH):
    for g in range(CH // 16):
      v = idx_ref[k, pl.ds(g * 16, 16)]
      j_ref[k, pl.ds(g * 16, 16)] = lax.shift_right_logical(v, 2)
      off_ref[k, pl.ds(g * 16, 16)] = lax.shift_left((v & 3), 5)


def _extract_chunk(g_ref, slot, off_ref, k, out_ref):
  lanes = lax.iota(jnp.int32, 16)
  svec = jnp.full((16,), slot, jnp.int32)
  kvec = jnp.full((16,), k, jnp.int32)

  def body(g, _):
    offv = off_ref[k, pl.ds(g * 16, 16)]
    rvec = lanes + g * 16
    dbase = rvec * EMB_DIM
    for c in range(EMB_DIM):
      v = plsc.load_gather(g_ref, [svec, rvec, offv + c])
      d = dbase + c
      plsc.store_scatter(out_ref,
                         [kvec, lax.shift_right_logical(d, 7), d & 127], v)
    return _

  lax.fori_loop(0, CH // 16, body, None)


def _tec_body(head_hbm, rel_hbm, tail_hbm, ent_hbm, relt_hbm,
              oh_hbm, or_hbm, ot_hbm,
              idx_v, j_v, off_h, off_r, off_t, g_v,
              rows_h, rows_r, rows_t, sem):
  wid = lax.axis_index("s") * NC + lax.axis_index("c")

  specs = ((head_hbm, off_h, ent_hbm, rows_h, oh_hbm),
           (rel_hbm, off_r, relt_hbm, rows_r, or_hbm),
           (tail_hbm, off_t, ent_hbm, rows_t, ot_hbm))
  for i_hbm, off_v, tab_hbm, rows_v, o_hbm in specs:
    pltpu.sync_copy(i_hbm.at[wid], idx_v)
    _prep_indices(idx_v, j_v, off_v)
    waits = [None, None]
    waits[0] = pltpu.async_copy(tab_hbm.at[j_v.at[0]], g_v.at[0], sem)
    for k in range(NCH):
      waits[k % 2].wait()
      if k + 1 < NCH:
        waits[(k + 1) % 2] = pltpu.async_copy(
            tab_hbm.at[j_v.at[k + 1]], g_v.at[(k + 1) % 2], sem)
      _extract_chunk(g_v, k % 2, off_v, k, rows_v)
    pltpu.sync_copy(rows_v, o_hbm.at[wid])


@jax.jit
def _transe_lookup(head, rel, tail, entity_table, relation_table):
  mesh = plsc.VectorSubcoreMesh(core_axis_name="c", subcore_axis_name="s")
  out_t = jax.ShapeDtypeStruct((NW, NCH, CH // 4, WIDE), jnp.float32)
  run = pl.kernel(
      _tec_body,
      out_type=(out_t, out_t, out_t),
      mesh=mesh,
      scratch_types=[
          pltpu.VMEM((NCH, CH), jnp.int32),
          pltpu.VMEM((NCH, CH), jnp.int32),
          pltpu.VMEM((NCH, CH), jnp.int32),
          pltpu.VMEM((NCH, CH), jnp.int32),
          pltpu.VMEM((NCH, CH), jnp.int32),
          pltpu.VMEM((2, CH, WIDE), jnp.float32),
          pltpu.VMEM((NCH, CH // 4, WIDE), jnp.float32),
          pltpu.VMEM((NCH, CH // 4, WIDE), jnp.float32),
          pltpu.VMEM((NCH, CH // 4, WIDE), jnp.float32),
          pltpu.SemaphoreType.DMA,
      ],
      compiler_params=pltpu.CompilerParams(needs_layout_passes=False),
  )
  ent_w = entity_table.reshape(NUM_ENTITIES // 4, WIDE)
  rel_w = relation_table.reshape(NUM_RELATIONS // 4, WIDE)
  h3 = head.reshape(NW, NCH, CH)
  r3 = rel.reshape(NW, NCH, CH)
  t3 = tail.reshape(NW, NCH, CH)
  oh, orr, ot = run(h3, r3, t3, ent_w, rel_w)
  return (oh.reshape(BATCH, EMB_DIM),
          orr.reshape(BATCH, EMB_DIM),
          ot.reshape(BATCH, EMB_DIM))


def kernel(head, rel, tail, entity_table, relation_table):
  return _transe_lookup(head, rel, tail, entity_table, relation_table)


# speedup vs baseline: 1.1073x; 1.1073x over previous
import functools

import jax
import jax.numpy as jnp
from jax import lax
from jax.experimental import pallas as pl
from jax.experimental.pallas import tpu as pltpu
from jax.experimental.pallas import tpu_sc as plsc

NUM_ENTITIES = 1000000
NUM_RELATIONS = 1000
EMB_DIM = 32
BATCH = 16384

NC = 2
NS = 16
NW = NC * NS
BPW = BATCH // NW
CH = 128
NCH = BPW // CH
WIDE = 128


def _prep_indices(idx_ref, j_ref, off_ref):
  for k in range(NCH):
    for g in range(CH // 16):
      v = idx_ref[k, pl.ds(g * 16, 16)]
      j_ref[k, pl.ds(g * 16, 16)] = lax.shift_right_logical(v, 2)
      off_ref[k, pl.ds(g * 16, 16)] = lax.shift_left((v & 3), 5)


def _extract_chunk(g_ref, slot, off_ref, k, rows_ref):
  lanes = lax.iota(jnp.int32, 16)
  svec = jnp.full((16,), slot, jnp.int32)

  def body(g, carry):
    offv = off_ref[k, pl.ds(g * 16, 16)]
    rvec = lanes + g * 16
    for c in range(EMB_DIM):
      v = plsc.load_gather(g_ref, [svec, rvec, offv + c])
      rows_ref[c >> 3, c & 7, pl.ds(k * CH + g * 16, 16)] = v
    return carry

  lax.fori_loop(0, CH // 16, body, None)


def _tec_body(head_hbm, rel_hbm, tail_hbm, ent_hbm, relt_hbm,
              oh_hbm, or_hbm, ot_hbm,
              idx_v, j_v, off_v, g_v, rel_v, rows_v, sem):
  wid = lax.axis_index("s") * NC + lax.axis_index("c")
  base = wid * BPW

  pltpu.sync_copy(relt_hbm, rel_v)

  for i_hbm, o_hbm in ((head_hbm, oh_hbm), (tail_hbm, ot_hbm)):
    pltpu.sync_copy(i_hbm.at[wid], idx_v)
    _prep_indices(idx_v, j_v, off_v)
    waits = [None, None]
    waits[0] = pltpu.async_copy(ent_hbm.at[j_v.at[0]], g_v.at[0], sem)
    for k in range(NCH):
      waits[k % 2].wait()
      if k + 1 < NCH:
        waits[(k + 1) % 2] = pltpu.async_copy(
            ent_hbm.at[j_v.at[k + 1]], g_v.at[(k + 1) % 2], sem)
      _extract_chunk(g_v, k % 2, off_v, k, rows_v)
    pltpu.sync_copy(rows_v, o_hbm.at[:, :, pl.ds(base, BPW)])

  pltpu.sync_copy(rel_hbm.at[wid], idx_v)

  def rel_step(g, carry):
    riv = idx_v[g // 8, pl.ds((g % 8) * 16, 16)]
    for tr in range(4):
      trv = jnp.full((16,), tr, jnp.int32)
      for e8 in range(8):
        v = plsc.load_gather(rel_v, [trv, jnp.full((16,), e8, jnp.int32), riv])
        rows_v[tr, e8, pl.ds(g * 16, 16)] = v
    return carry

  lax.fori_loop(0, BPW // 16, rel_step, None)
  pltpu.sync_copy(rows_v, or_hbm.at[:, :, pl.ds(base, BPW)])


@jax.jit
def _transe_lookup(head, rel, tail, entity_table, relation_table):
  mesh = plsc.VectorSubcoreMesh(core_axis_name="c", subcore_axis_name="s")
  out_t = jax.ShapeDtypeStruct((4, 8, BATCH), jnp.float32)
  run = pl.kernel(
      _tec_body,
      out_type=(out_t, out_t, out_t),
      mesh=mesh,
      scratch_types=[
          pltpu.VMEM((NCH, CH), jnp.int32),
          pltpu.VMEM((NCH, CH), jnp.int32),
          pltpu.VMEM((NCH, CH), jnp.int32),
          pltpu.VMEM((2, CH, WIDE), jnp.float32),
          pltpu.VMEM((4, 8, NUM_RELATIONS), jnp.float32),
          pltpu.VMEM((4, 8, BPW), jnp.float32),
          pltpu.SemaphoreType.DMA,
      ],
      compiler_params=pltpu.CompilerParams(needs_layout_passes=False),
  )
  ent_w = entity_table.reshape(NUM_ENTITIES // 4, WIDE)
  rel4 = relation_table.T.reshape(4, 8, NUM_RELATIONS)
  h3 = head.reshape(NW, NCH, CH)
  r3 = rel.reshape(NW, NCH, CH)
  t3 = tail.reshape(NW, NCH, CH)
  oh, orr, ot = run(h3, r3, t3, ent_w, rel4)
  return (oh.reshape(EMB_DIM, BATCH).T,
          orr.reshape(EMB_DIM, BATCH).T,
          ot.reshape(EMB_DIM, BATCH).T)


def kernel(head, rel, tail, entity_table, relation_table):
  return _transe_lookup(head, rel, tail, entity_table, relation_table)


# speedup vs baseline: 1.1149x; 1.0069x over previous
import jax
import jax.numpy as jnp
from jax import lax
from jax.experimental import pallas as pl
from jax.experimental.pallas import tpu as pltpu
from jax.experimental.pallas import tpu_sc as plsc

NUM_ENTITIES = 1000000
NUM_RELATIONS = 1000
EMB_DIM = 32
BATCH = 16384

NC = 2
NS = 16
NW = NC * NS
BPW = BATCH // NW


def _tec_body(head_hbm, rel_hbm, tail_hbm, ent_hbm, relt_hbm,
              oh_hbm, or_hbm, ot_hbm,
              hidx_v, ridx_v, tidx_v, hrows_v, rrows_v, trows_v, sem):
  wid = lax.axis_index("s") * NC + lax.axis_index("c")
  base = wid * BPW

  pltpu.sync_copy(head_hbm.at[pl.ds(base, BPW)], hidx_v)
  pltpu.sync_copy(rel_hbm.at[pl.ds(base, BPW)], ridx_v)
  pltpu.sync_copy(tail_hbm.at[pl.ds(base, BPW)], tidx_v)

  ch = pltpu.async_copy(ent_hbm.at[hidx_v], hrows_v, sem)
  cr = pltpu.async_copy(relt_hbm.at[ridx_v], rrows_v, sem)
  ct = pltpu.async_copy(ent_hbm.at[tidx_v], trows_v, sem)
  ch.wait()
  cr.wait()
  ct.wait()

  pltpu.sync_copy(hrows_v, oh_hbm.at[pl.ds(base, BPW)])
  pltpu.sync_copy(rrows_v, or_hbm.at[pl.ds(base, BPW)])
  pltpu.sync_copy(trows_v, ot_hbm.at[pl.ds(base, BPW)])


@jax.jit
def _transe_lookup(head, rel, tail, entity_table, relation_table):
  mesh = plsc.VectorSubcoreMesh(core_axis_name="c", subcore_axis_name="s")
  out_t = jax.ShapeDtypeStruct((BATCH, EMB_DIM), jnp.float32)
  run = pl.kernel(
      _tec_body,
      out_type=(out_t, out_t, out_t),
      mesh=mesh,
      scratch_types=[
          pltpu.VMEM((BPW,), jnp.int32),
          pltpu.VMEM((BPW,), jnp.int32),
          pltpu.VMEM((BPW,), jnp.int32),
          pltpu.VMEM((BPW, EMB_DIM), jnp.float32),
          pltpu.VMEM((BPW, EMB_DIM), jnp.float32),
          pltpu.VMEM((BPW, EMB_DIM), jnp.float32),
          pltpu.SemaphoreType.DMA,
      ],
      compiler_params=pltpu.CompilerParams(use_tc_tiling_on_sc=False),
  )
  return run(head, rel, tail, entity_table, relation_table)


def kernel(head, rel, tail, entity_table, relation_table):
  return _transe_lookup(head, rel, tail, entity_table, relation_table)
